# Initial kernel scaffold; baseline (speedup 1.0000x reference)
import jax
import jax.numpy as jnp
from jax.experimental import pallas as pl


def kernel(x, W_follows, b_follows, W_likes, b_likes, loop_weight, h_bias, edge_index_follows, edge_index_likes):
    raise NotImplementedError("write your pallas kernel here")


# speedup vs baseline: 1.3163x; 1.3163x over previous
import jax
import jax.numpy as jnp
from jax import lax
from jax.experimental import pallas as pl
from jax.experimental.pallas import tpu as pltpu
from jax.experimental.pallas import tpu_sc as plsc

_N = 50000
_D = 128
_E = 400000
_EP = 409600
_SH = _EP // 16
_BB = 6400
_NBATCH = _SH // _BB
_STEPS = _BB // 16
_CH = 8448
_NCHUNK = 6
_CPC = 3
_MPAD = _NCHUNK * _CH
_ROWS_T = _CH // 16
_DUMMY = _CH
_ACC = _CH + 1
_G = 128
_NGB = _BB // _G + 1


def _sc_body(x_hbm, src_f, dst_f, src_l, dst_l,
             a_f, deg_f, a_l, deg_l,
             acc_sp, dsum_sp, src_fb, ldst_fb, idx_row, src_eb, dst_eb,
             rows_v, zbuf, ones_g, dout_v, sem):
    c = lax.axis_index("c")
    s = lax.axis_index("s")

    def _init(i, carry):
        for k in range(8):
            zbuf[i, pl.ds(k * 16, 16)] = jnp.zeros((16,), jnp.float32)
        return carry
    lax.fori_loop(0, 16, _init, 0)
    for k in range(8):
        ones_g[pl.ds(k * 16, 16)] = jnp.ones((16,), jnp.float32)

    iota = lax.iota(jnp.int32, 16)
    zeros_i = jnp.zeros((16,), jnp.int32)
    dummy_i = jnp.full((16,), _DUMMY, jnp.int32)

    for et, (src_h, dst_h, a_h, deg_h) in enumerate(
            ((src_f, dst_f, a_f, deg_f), (src_l, dst_l, a_l, deg_l))):
        for p in range(_CPC):
            q = _CPC * c + p
            lo = q * _CH
            lo_v = lax.cond(
                c == 0,
                lambda: jnp.full((16,), p * _CH, jnp.int32),
                lambda: jnp.full((16,), (_CPC + p) * _CH, jnp.int32))
            hi_v = lo_v + _CH

            for r in range(_ROWS_T // 16):
                pltpu.sync_copy(zbuf, acc_sp.at[pl.ds(s * _ROWS_T + r * 16, 16)])
            for r in range(4):
                pltpu.sync_copy(zbuf.at[0], dsum_sp.at[pl.ds(s * _ROWS_T + r * 128, 128)])
            pltpu.sync_copy(zbuf.at[0, pl.ds(0, 16)],
                            dsum_sp.at[pl.ds(s * _ROWS_T + 512, 16)])
            plsc.subcore_barrier()

            for b in range(_NBATCH):
                ebase = s * _SH + b * _BB
                pltpu.sync_copy(src_h.at[pl.ds(ebase, _BB)], src_eb)
                pltpu.sync_copy(dst_h.at[pl.ds(ebase, _BB)], dst_eb)

                def _step(j, carry):
                    cnt, cnt_v = carry
                    off = j * 16
                    dvec = dst_eb[pl.ds(off, 16)]
                    svec = src_eb[pl.ds(off, 16)]
                    m = (dvec >= lo_v) & (dvec < hi_v)
                    plsc.store_compressed(src_fb.at[pl.ds(cnt, 16)],
                                          svec, mask=m)
                    plsc.store_compressed(ldst_fb.at[pl.ds(cnt, 16)],
                                          dvec - lo_v, mask=m)
                    pc = plsc.all_reduce_population_count(m)
                    return cnt + pc[0], cnt_v + pc
                cnt, cnt_v = lax.fori_loop(
                    0, _STEPS, _step,
                    (jnp.int32(0), jnp.zeros((16,), jnp.int32)))

                cnt_pad_v = lax.shift_left(
                    lax.shift_right_logical(cnt_v + 127, 7), 7)
                for k in range(8):
                    mp = (cnt_v + (k * 16) + iota) < cnt_pad_v
                    plsc.store_compressed(src_fb.at[pl.ds(cnt + k * 16, 16)],
                                          zeros_i, mask=mp)
                    plsc.store_compressed(ldst_fb.at[pl.ds(cnt + k * 16, 16)],
                                          dummy_i, mask=mp)

                ng = lax.shift_right_logical(cnt + 127, 7)

                def _flush(g, carry):
                    for k in range(_G // 16):
                        idx_row[0, pl.ds(k * 16, 16)] = (
                            ldst_fb[pl.ds(g * _G + k * 16, 16)])
                    pltpu.async_copy(x_hbm.at[src_fb.at[pl.ds(g * _G, _G)]],
                                     rows_v, sem).wait()
                    pltpu.sync_copy(rows_v, acc_sp.at[idx_row.at[0]], add=True)
                    pltpu.sync_copy(ones_g, dsum_sp.at[idx_row.at[0]], add=True)
                    return carry
                lax.fori_loop(0, ng, _flush, 0)

            plsc.subcore_barrier()

            pltpu.sync_copy(acc_sp.at[pl.ds(s * _ROWS_T, _ROWS_T)],
                            a_h.at[pl.ds(lo + s * _ROWS_T, _ROWS_T)])
            pltpu.sync_copy(dsum_sp.at[pl.ds(s * _ROWS_T, _ROWS_T)], dout_v)
            pltpu.sync_copy(dout_v, deg_h.at[pl.ds(lo + s * _ROWS_T, _ROWS_T)])
            plsc.subcore_barrier()


_sc_segsum = pl.kernel(
    _sc_body,
    out_type=[
        jax.ShapeDtypeStruct((_MPAD, _D), jnp.float32),
        jax.ShapeDtypeStruct((_MPAD,), jnp.float32),
        jax.ShapeDtypeStruct((_MPAD, _D), jnp.float32),
        jax.ShapeDtypeStruct((_MPAD,), jnp.float32),
    ],
    mesh=plsc.VectorSubcoreMesh(core_axis_name="c", subcore_axis_name="s"),
    compiler_params=pltpu.CompilerParams(needs_layout_passes=False),
    scratch_types=[
        pltpu.VMEM_SHARED((_ACC, _D), jnp.float32),
        pltpu.VMEM_SHARED((_ACC,), jnp.float32),
        pltpu.VMEM((_NGB * _G,), jnp.int32),
        pltpu.VMEM((_NGB * _G,), jnp.int32),
        pltpu.VMEM((1, _G), jnp.int32),
        pltpu.VMEM((_BB,), jnp.int32),
        pltpu.VMEM((_BB,), jnp.int32),
        pltpu.VMEM((_G, _D), jnp.float32),
        pltpu.VMEM((16, _D), jnp.float32),
        pltpu.VMEM((_G,), jnp.float32),
        pltpu.VMEM((_ROWS_T,), jnp.float32),
        pltpu.SemaphoreType.DMA,
    ],
)


_BM = 256


def _tc_body(af_ref, al_ref, x_ref, df_ref, dl_ref, w_ref, bf_ref, bl_ref,
             hb_ref, out_ref):
    df = df_ref[...]
    dl = dl_ref[...]
    rf = 1.0 / jnp.maximum(df, 1.0)
    rl = 1.0 / jnp.maximum(dl, 1.0)
    cat = jnp.concatenate(
        [af_ref[...] * rf, al_ref[...] * rl, x_ref[...]], axis=1)
    y = jnp.dot(cat, w_ref[...], preferred_element_type=jnp.float32)
    y = y + jnp.where(df > 0, 1.0, 0.0) * bf_ref[...]
    y = y + jnp.where(dl > 0, 1.0, 0.0) * bl_ref[...]
    out_ref[...] = y + hb_ref[...]


_tc_combine = pl.pallas_call(
    _tc_body,
    grid=(_MPAD // _BM,),
    in_specs=[
        pl.BlockSpec((_BM, _D), lambda i: (i, 0)),
        pl.BlockSpec((_BM, _D), lambda i: (i, 0)),
        pl.BlockSpec((_BM, _D), lambda i: (i, 0)),
        pl.BlockSpec((_BM, 1), lambda i: (i, 0)),
        pl.BlockSpec((_BM, 1), lambda i: (i, 0)),
        pl.BlockSpec((3 * _D, _D), lambda i: (0, 0)),
        pl.BlockSpec((1, _D), lambda i: (0, 0)),
        pl.BlockSpec((1, _D), lambda i: (0, 0)),
        pl.BlockSpec((1, _D), lambda i: (0, 0)),
    ],
    out_specs=pl.BlockSpec((_BM, _D), lambda i: (i, 0)),
    out_shape=jax.ShapeDtypeStruct((_MPAD, _D), jnp.float32),
)


@jax.jit
def kernel(x, W_follows, b_follows, W_likes, b_likes, loop_weight, h_bias,
           edge_index_follows, edge_index_likes):
    pad = _EP - _E
    pad_src = jnp.zeros((pad,), jnp.int32)
    pad_dst = jnp.full((pad,), _MPAD - 1, jnp.int32)
    src_f = jnp.concatenate([edge_index_follows[0], pad_src])
    dst_f = jnp.concatenate([edge_index_follows[1], pad_dst])
    src_l = jnp.concatenate([edge_index_likes[0], pad_src])
    dst_l = jnp.concatenate([edge_index_likes[1], pad_dst])

    a_f, deg_f, a_l, deg_l = _sc_segsum(x, src_f, dst_f, src_l, dst_l)

    xp = jnp.concatenate([x, jnp.zeros((_MPAD - _N, _D), jnp.float32)])
    w_cat = jnp.concatenate([W_follows, W_likes, loop_weight], axis=0)
    out = _tc_combine(a_f, a_l, xp, deg_f[:, None], deg_l[:, None], w_cat,
                      b_follows[None, :], b_likes[None, :], h_bias[None, :])
    return out[:_N]


# speedup vs baseline: 1.3920x; 1.0575x over previous
import jax
import jax.numpy as jnp
from jax import lax
from jax.experimental import pallas as pl
from jax.experimental.pallas import tpu as pltpu
from jax.experimental.pallas import tpu_sc as plsc

_N = 50000
_D = 128
_E = 400000
_EP = 409600
_SH = _EP // 16
_BB = 6400
_NBATCH = _SH // _BB
_STEPS = _BB // 16
_CH = 8448
_NCHUNK = 6
_CPC = 3
_MPAD = _NCHUNK * _CH
_ROWS_T = _CH // 16
_DUMMY = _CH
_ACC = _CH + 1
_G = 128
_NGB = _BB // _G + 1


def _sc_body(x_hbm, src_f, dst_f, src_l, dst_l,
             a_f, deg_f, a_l, deg_l,
             acc_sp, dsum_sp, src_fb, ldst_fb, idx_row, src_eb, dst_eb,
             rows_v, rows_w, zbuf, ones_g, dout_v, sem, sem2):
    c = lax.axis_index("c")
    s = lax.axis_index("s")

    def _init(i, carry):
        for k in range(8):
            zbuf[i, pl.ds(k * 16, 16)] = jnp.zeros((16,), jnp.float32)
        return carry
    lax.fori_loop(0, 8, _init, 0)
    for k in range(8):
        ones_g[pl.ds(k * 16, 16)] = jnp.ones((16,), jnp.float32)

    iota = lax.iota(jnp.int32, 16)
    zeros_i = jnp.zeros((16,), jnp.int32)
    dummy_i = jnp.full((16,), _DUMMY, jnp.int32)

    for et, (src_h, dst_h, a_h, deg_h) in enumerate(
            ((src_f, dst_f, a_f, deg_f), (src_l, dst_l, a_l, deg_l))):
        for p in range(_CPC):
            q = _CPC * c + p
            lo = q * _CH
            lo_v = lax.cond(
                c == 0,
                lambda: jnp.full((16,), p * _CH, jnp.int32),
                lambda: jnp.full((16,), (_CPC + p) * _CH, jnp.int32))
            hi_v = lo_v + _CH

            for r in range(_ROWS_T // 8):
                pltpu.sync_copy(zbuf, acc_sp.at[pl.ds(s * _ROWS_T + r * 8, 8)])
            for r in range(4):
                pltpu.sync_copy(zbuf.at[0], dsum_sp.at[pl.ds(s * _ROWS_T + r * 128, 128)])
            pltpu.sync_copy(zbuf.at[0, pl.ds(0, 16)],
                            dsum_sp.at[pl.ds(s * _ROWS_T + 512, 16)])
            plsc.subcore_barrier()

            for b in range(_NBATCH):
                ebase = s * _SH + b * _BB
                pltpu.sync_copy(src_h.at[pl.ds(ebase, _BB)], src_eb)
                pltpu.sync_copy(dst_h.at[pl.ds(ebase, _BB)], dst_eb)

                def _step(j, carry):
                    cnt, cnt_v = carry
                    off = j * 16
                    dvec = dst_eb[pl.ds(off, 16)]
                    svec = src_eb[pl.ds(off, 16)]
                    m = (dvec >= lo_v) & (dvec < hi_v)
                    plsc.store_compressed(src_fb.at[pl.ds(cnt, 16)],
                                          svec, mask=m)
                    plsc.store_compressed(ldst_fb.at[pl.ds(cnt, 16)],
                                          dvec - lo_v, mask=m)
                    pc = plsc.all_reduce_population_count(m)
                    return cnt + pc[0], cnt_v + pc
                cnt, cnt_v = lax.fori_loop(
                    0, _STEPS, _step,
                    (jnp.int32(0), jnp.zeros((16,), jnp.int32)))

                cnt_pad_v = lax.shift_left(
                    lax.shift_right_logical(cnt_v + 127, 7), 7)
                for k in range(8):
                    mp = (cnt_v + (k * 16) + iota) < cnt_pad_v
                    plsc.store_compressed(src_fb.at[pl.ds(cnt + k * 16, 16)],
                                          zeros_i, mask=mp)
                    plsc.store_compressed(ldst_fb.at[pl.ds(cnt + k * 16, 16)],
                                          dummy_i, mask=mp)

                ng = lax.shift_right_logical(cnt + 127, 7)

                def _gather(g, buf, gsem):
                    pltpu.async_copy(
                        x_hbm.at[src_fb.at[pl.ds(g * _G, _G)]], buf, gsem)

                def _gwait(g, buf, gsem):
                    pltpu.make_async_copy(
                        x_hbm.at[src_fb.at[pl.ds(g * _G, _G)]], buf,
                        gsem).wait()

                def _scatter(g, buf, row):
                    for k in range(_G // 16):
                        idx_row[row, pl.ds(k * 16, 16)] = (
                            ldst_fb[pl.ds(g * _G + k * 16, 16)])
                    pltpu.sync_copy(buf, acc_sp.at[idx_row.at[row]], add=True)
                    pltpu.sync_copy(ones_g, dsum_sp.at[idx_row.at[row]],
                                    add=True)

                @pl.when(ng > 0)
                def _():
                    _gather(0, rows_v, sem)

                def _flush2(gg, carry):
                    g0 = 2 * gg

                    @pl.when(g0 + 1 < ng)
                    def _():
                        _gather(g0 + 1, rows_w, sem2)
                    _gwait(g0, rows_v, sem)
                    _scatter(g0, rows_v, 0)

                    @pl.when(g0 + 2 < ng)
                    def _():
                        _gather(g0 + 2, rows_v, sem)

                    @pl.when(g0 + 1 < ng)
                    def _():
                        _gwait(g0 + 1, rows_w, sem2)
                        _scatter(g0 + 1, rows_w, 1)
                    return carry
                lax.fori_loop(0, lax.shift_right_logical(ng + 1, 1),
                              _flush2, 0)

            plsc.subcore_barrier()

            pltpu.sync_copy(acc_sp.at[pl.ds(s * _ROWS_T, _ROWS_T)],
                            a_h.at[pl.ds(lo + s * _ROWS_T, _ROWS_T)])
            pltpu.sync_copy(dsum_sp.at[pl.ds(s * _ROWS_T, _ROWS_T)], dout_v)
            pltpu.sync_copy(dout_v, deg_h.at[pl.ds(lo + s * _ROWS_T, _ROWS_T)])
            plsc.subcore_barrier()


_sc_segsum = pl.kernel(
    _sc_body,
    out_type=[
        jax.ShapeDtypeStruct((_MPAD, _D), jnp.float32),
        jax.ShapeDtypeStruct((_MPAD,), jnp.float32),
        jax.ShapeDtypeStruct((_MPAD, _D), jnp.float32),
        jax.ShapeDtypeStruct((_MPAD,), jnp.float32),
    ],
    mesh=plsc.VectorSubcoreMesh(core_axis_name="c", subcore_axis_name="s"),
    compiler_params=pltpu.CompilerParams(needs_layout_passes=False),
    scratch_types=[
        pltpu.VMEM_SHARED((_ACC, _D), jnp.float32),
        pltpu.VMEM_SHARED((_ACC,), jnp.float32),
        pltpu.VMEM((_NGB * _G,), jnp.int32),
        pltpu.VMEM((_NGB * _G,), jnp.int32),
        pltpu.VMEM((2, _G), jnp.int32),
        pltpu.VMEM((_BB,), jnp.int32),
        pltpu.VMEM((_BB,), jnp.int32),
        pltpu.VMEM((_G, _D), jnp.float32),
        pltpu.VMEM((_G, _D), jnp.float32),
        pltpu.VMEM((8, _D), jnp.float32),
        pltpu.VMEM((_G,), jnp.float32),
        pltpu.VMEM((_ROWS_T,), jnp.float32),
        pltpu.SemaphoreType.DMA,
        pltpu.SemaphoreType.DMA,
    ],
)


_BM = 256


def _tc_body(af_ref, al_ref, x_ref, df_ref, dl_ref, w_ref, bf_ref, bl_ref,
             hb_ref, out_ref):
    df = df_ref[...]
    dl = dl_ref[...]
    rf = 1.0 / jnp.maximum(df, 1.0)
    rl = 1.0 / jnp.maximum(dl, 1.0)
    cat = jnp.concatenate(
        [af_ref[...] * rf, al_ref[...] * rl, x_ref[...]], axis=1)
    y = jnp.dot(cat, w_ref[...], preferred_element_type=jnp.float32)
    y = y + jnp.where(df > 0, 1.0, 0.0) * bf_ref[...]
    y = y + jnp.where(dl > 0, 1.0, 0.0) * bl_ref[...]
    out_ref[...] = y + hb_ref[...]


_tc_combine = pl.pallas_call(
    _tc_body,
    grid=(_MPAD // _BM,),
    in_specs=[
        pl.BlockSpec((_BM, _D), lambda i: (i, 0)),
        pl.BlockSpec((_BM, _D), lambda i: (i, 0)),
        pl.BlockSpec((_BM, _D), lambda i: (i, 0)),
        pl.BlockSpec((_BM, 1), lambda i: (i, 0)),
        pl.BlockSpec((_BM, 1), lambda i: (i, 0)),
        pl.BlockSpec((3 * _D, _D), lambda i: (0, 0)),
        pl.BlockSpec((1, _D), lambda i: (0, 0)),
        pl.BlockSpec((1, _D), lambda i: (0, 0)),
        pl.BlockSpec((1, _D), lambda i: (0, 0)),
    ],
    out_specs=pl.BlockSpec((_BM, _D), lambda i: (i, 0)),
    out_shape=jax.ShapeDtypeStruct((_MPAD, _D), jnp.float32),
)


@jax.jit
def kernel(x, W_follows, b_follows, W_likes, b_likes, loop_weight, h_bias,
           edge_index_follows, edge_index_likes):
    pad = _EP - _E
    pad_src = jnp.zeros((pad,), jnp.int32)
    pad_dst = jnp.full((pad,), _MPAD - 1, jnp.int32)
    src_f = jnp.concatenate([edge_index_follows[0], pad_src])
    dst_f = jnp.concatenate([edge_index_follows[1], pad_dst])
    src_l = jnp.concatenate([edge_index_likes[0], pad_src])
    dst_l = jnp.concatenate([edge_index_likes[1], pad_dst])

    a_f, deg_f, a_l, deg_l = _sc_segsum(x, src_f, dst_f, src_l, dst_l)

    xp = jnp.concatenate([x, jnp.zeros((_MPAD - _N, _D), jnp.float32)])
    w_cat = jnp.concatenate([W_follows, W_likes, loop_weight], axis=0)
    out = _tc_combine(a_f, a_l, xp, deg_f[:, None], deg_l[:, None], w_cat,
                      b_follows[None, :], b_likes[None, :], h_bias[None, :])
    return out[:_N]
